# baseline (speedup 1.0000x reference)
import functools

import jax
import jax.numpy as jnp
from jax.experimental import pallas as pl
from jax.experimental.pallas import tpu as pltpu

_K = 4
_PAD = 2
_EPS = 1e-5
_SLOPE = 0.2

_CFGS = [
    (3, 64, 2),
    (64, 128, 2),
    (128, 256, 2),
    (256, 512, 1),
    (512, 1, 1),
]


def _spatial(h, w):
    shapes = []
    ih, iw = h, w
    for (_ci, _co, stride) in _CFGS:
        oh = (ih + 2 * _PAD - _K) // stride + 1
        ow = (iw + 2 * _PAD - _K) // stride + 1
        shapes.append((oh, ow))
        ih, iw = oh, ow
    return shapes


def _padded_dim(o, even):
    d = o + 2 * _PAD
    return d + (d % 2) if even else d


def _disc_kernel(cols_ref, w1_ref, w2_ref, w3_ref, w4_ref, w5t_ref, b5_ref,
                 s1_ref, s2_ref, s3_ref, s4_ref,
                 o_ref,
                 p1_00, p1_01, p1_10, p1_11, act2, act3, act4,
                 *, bsz, shapes):
    (oh1, ow1), (oh2, ow2), (oh3, ow3), (oh4, ow4), (oh5, ow5) = shapes
    ph1 = ((p1_00, p1_01), (p1_10, p1_11))

    def norm_lrelu(y, s_ref, p, c):
        z = jnp.concatenate([y.astype(jnp.bfloat16),
                             (y * y).astype(jnp.bfloat16)], axis=1)
        st = jnp.dot(s_ref[...], z, preferred_element_type=jnp.float32) / p
        mean = st[:, :c]
        var = st[:, c:] - mean * mean
        scale = jax.lax.rsqrt(var + _EPS)
        y = y.reshape(bsz, p, c)
        yn = (y - mean[:, None, :]) * scale[:, None, :]
        return jnp.where(yn > 0, yn, _SLOPE * yn)

    def store_padded(buf, y, oh, ow, c):
        hp, wp = buf.shape[1], buf.shape[2]
        buf[:, :_PAD] = jnp.zeros_like(buf[:, :_PAD])
        buf[:, _PAD + oh:] = jnp.zeros_like(buf[:, _PAD + oh:])
        buf[:, _PAD:_PAD + oh, :_PAD] = (
            jnp.zeros_like(buf[:, _PAD:_PAD + oh, :_PAD]))
        buf[:, _PAD:_PAD + oh, _PAD + ow:] = (
            jnp.zeros_like(buf[:, _PAD:_PAD + oh, _PAD + ow:]))
        buf[:, _PAD:_PAD + oh, _PAD:_PAD + ow] = (
            y.reshape(bsz, oh, ow, c).astype(buf.dtype))

    def conv_taps(buf, w_ref, cin, stride, oh, ow):
        p = oh * ow
        if stride == 2 and not isinstance(buf, tuple):
            hph = buf.shape[1] // 2
            wph = buf.shape[2] // 2
            phases = [[buf[:, pl.ds(a, hph, stride=2),
                          pl.ds(b, wph, stride=2), :]
                       for b in range(2)] for a in range(2)]
        y = None
        for i in range(_K):
            for j in range(_K):
                t = i * _K + j
                if stride == 2 and isinstance(buf, tuple):
                    xs = buf[i % 2][j % 2][:, i // 2:i // 2 + oh,
                                           j // 2:j // 2 + ow, :]
                elif stride == 2:
                    xs = phases[i % 2][j % 2][:, i // 2:i // 2 + oh,
                                              j // 2:j // 2 + ow, :]
                else:
                    xs = buf[:, i:i + oh, j:j + ow, :]
                xt = xs.reshape(bsz * p, cin).astype(jnp.bfloat16)
                d = jnp.dot(xt, w_ref[t * cin:(t + 1) * cin, :],
                            preferred_element_type=jnp.float32)
                y = d if y is None else y + d
        return y

    p1 = oh1 * ow1
    y = jnp.dot(cols_ref[...].reshape(bsz * p1, _K * _K * 3), w1_ref[...],
                preferred_element_type=jnp.float32)
    y = norm_lrelu(y, s1_ref, p1, 64)
    y = y.reshape(bsz, p1, 64)
    off = 0
    for a in (0, 1):
        nh = (oh1 - a + 1) // 2
        for b in (0, 1):
            nw = (ow1 - b + 1) // 2
            buf = ph1[a][b]
            buf[...] = jnp.zeros_like(buf)
            buf[:, 1:1 + nh, 1:1 + nw, :] = (
                y[:, off:off + nh * nw, :].reshape(bsz, nh, nw, 64)
                .astype(jnp.bfloat16))
            off += nh * nw

    y = conv_taps(ph1, w2_ref, 64, 2, oh2, ow2)
    y = norm_lrelu(y, s2_ref, oh2 * ow2, 128)
    store_padded(act2, y, oh2, ow2, 128)

    y = conv_taps(act2, w3_ref, 128, 2, oh3, ow3)
    y = norm_lrelu(y, s3_ref, oh3 * ow3, 256)
    store_padded(act3, y, oh3, ow3, 256)

    y = conv_taps(act3, w4_ref, 256, 1, oh4, ow4)
    y = norm_lrelu(y, s4_ref, oh4 * ow4, 512)
    store_padded(act4, y, oh4, ow4, 512)

    hp4, wp4 = act4.shape[1], act4.shape[2]
    g = jnp.dot(act4[...].reshape(bsz * hp4 * wp4, 512), w5t_ref[...],
                preferred_element_type=jnp.float32)
    g4 = g.reshape(bsz, hp4, wp4, _K * _K)
    lane = jax.lax.broadcasted_iota(jnp.int32, (1, 1, 1, _K * _K), 3)
    acc = jnp.zeros((bsz, oh5, ow5, _K * _K), jnp.float32)
    for i in range(_K):
        for j in range(_K):
            t = i * _K + j
            acc = acc + jnp.where(lane == t,
                                  g4[:, i:i + oh5, j:j + ow5, :], 0.0)
    y5 = jnp.sum(acc, axis=-1) + b5_ref[0, 0]
    o_ref[...] = jax.nn.sigmoid(y5)


def kernel(x, w1, b1, w2, b2, w3, b3, w4, b4, w5, b5):
    n, cin, h, w = x.shape
    assert cin == 3
    shapes = _spatial(h, w)
    (oh1, ow1), (oh2, ow2), (oh3, ow3), (oh4, ow4), (oh5, ow5) = shapes
    p1 = oh1 * ow1

    xt = jnp.transpose(x, (0, 2, 3, 1))
    xp = jnp.pad(xt, ((0, 0), (_PAD, _PAD), (_PAD, _PAD), (0, 0)))
    taps = []
    for i in range(_K):
        for j in range(_K):
            taps.append(xp[:, i:i + 2 * oh1 - 1:2, j:j + 2 * ow1 - 1:2, :])
    cols = jnp.concatenate(taps, axis=-1).reshape(n, oh1, ow1, _K * _K * 3)
    cols = jnp.concatenate(
        [cols[:, a::2, b::2, :].reshape(n, -1, _K * _K * 3)
         for a in (0, 1) for b in (0, 1)], axis=1)
    cols = cols.astype(jnp.bfloat16)

    w1m = w1.reshape(_K * _K * 3, 64).astype(jnp.bfloat16)
    w2m = w2.reshape(_K * _K * 64, 128).astype(jnp.bfloat16)
    w3m = w3.reshape(_K * _K * 128, 256).astype(jnp.bfloat16)
    w4m = w4.reshape(_K * _K * 256, 512).astype(jnp.bfloat16)
    w5t = jnp.transpose(w5.reshape(_K * _K, 512), (1, 0)).astype(jnp.bfloat16)
    b5s = b5.reshape(1, 1).astype(jnp.float32)

    bsz = max(d for d in (16, 8, 4, 2, 1) if n % d == 0)
    grid = (n // bsz,)

    def sel(p):
        r = jnp.arange(bsz, dtype=jnp.int32)[:, None]
        q = jnp.arange(bsz * p, dtype=jnp.int32)[None, :] // p
        return (r == q).astype(jnp.bfloat16)

    sels = [sel(oh * ow) for (oh, ow) in shapes[:4]]

    r1 = max(oh2 + 1, 1 + (oh1 + 1) // 2) + 1
    c1 = max(ow2 + 1, 1 + (ow1 + 1) // 2) + 1
    scratch_shapes = [
        pltpu.VMEM((bsz, r1, c1, 64), jnp.bfloat16),
        pltpu.VMEM((bsz, r1, c1, 64), jnp.bfloat16),
        pltpu.VMEM((bsz, r1, c1, 64), jnp.bfloat16),
        pltpu.VMEM((bsz, r1, c1, 64), jnp.bfloat16),
        pltpu.VMEM((bsz, _padded_dim(oh2, True), _padded_dim(ow2, True), 128),
                   jnp.float32),
        pltpu.VMEM((bsz, _padded_dim(oh3, False), _padded_dim(ow3, False),
                    256), jnp.bfloat16),
        pltpu.VMEM((bsz, _padded_dim(oh4, False), _padded_dim(ow4, False),
                    512), jnp.bfloat16),
    ]

    body = functools.partial(_disc_kernel, bsz=bsz, shapes=tuple(shapes))

    in_specs = (
        [pl.BlockSpec((bsz, p1, _K * _K * 3), lambda i: (i, 0, 0)),
         pl.BlockSpec(w1m.shape, lambda i: (0, 0)),
         pl.BlockSpec(w2m.shape, lambda i: (0, 0)),
         pl.BlockSpec(w3m.shape, lambda i: (0, 0)),
         pl.BlockSpec(w4m.shape, lambda i: (0, 0)),
         pl.BlockSpec(w5t.shape, lambda i: (0, 0)),
         pl.BlockSpec(b5s.shape, lambda i: (0, 0))]
        + [pl.BlockSpec(s.shape, lambda i: (0, 0)) for s in sels])

    cins = [3, 64, 128, 256, 512]
    couts = [64, 128, 256, 512, 16]
    flops = sum(2 * n * oh * ow * (_K * _K * ci) * co
                for (oh, ow), ci, co in zip(shapes, cins, couts))
    transcendentals = n * (64 + 128 + 256 + 512) + n * oh5 * ow5
    bytes_accessed = (cols.size * 2 + w1m.size * 2 + w2m.size * 2
                      + w3m.size * 2 + w4m.size * 2 + w5t.size * 2
                      + n * oh5 * ow5 * 4)

    out = pl.pallas_call(
        body,
        out_shape=jax.ShapeDtypeStruct((n, oh5, ow5), jnp.float32),
        grid=grid,
        in_specs=in_specs,
        out_specs=pl.BlockSpec((bsz, oh5, ow5), lambda i: (i, 0, 0)),
        scratch_shapes=scratch_shapes,
        compiler_params=pltpu.CompilerParams(
            dimension_semantics=("parallel",),
            vmem_limit_bytes=56 * 1024 * 1024,
        ),
        cost_estimate=pl.CostEstimate(
            flops=flops, transcendentals=transcendentals,
            bytes_accessed=bytes_accessed),
    )(cols, w1m, w2m, w3m, w4m, w5t, b5s, *sels)

    return out[:, None, :, :]


# speedup vs baseline: 1.0004x; 1.0004x over previous
import functools

import jax
import jax.numpy as jnp
from jax.experimental import pallas as pl
from jax.experimental.pallas import tpu as pltpu

_K = 4
_PAD = 2
_EPS = 1e-5
_SLOPE = 0.2

_CFGS = [
    (3, 64, 2),
    (64, 128, 2),
    (128, 256, 2),
    (256, 512, 1),
    (512, 1, 1),
]


def _spatial(h, w):
    shapes = []
    ih, iw = h, w
    for (_ci, _co, stride) in _CFGS:
        oh = (ih + 2 * _PAD - _K) // stride + 1
        ow = (iw + 2 * _PAD - _K) // stride + 1
        shapes.append((oh, ow))
        ih, iw = oh, ow
    return shapes


def _padded_dim(o, even):
    d = o + 2 * _PAD
    return d + (d % 2) if even else d


def _disc_kernel(cols_ref, w1_ref, w2_ref, w3_ref, w4_ref, w5t_ref, b5_ref,
                 s1_ref, s2_ref, s3_ref, s4_ref,
                 o_ref,
                 act1, act2, act3, act4,
                 *, bsz, shapes):
    (oh1, ow1), (oh2, ow2), (oh3, ow3), (oh4, ow4), (oh5, ow5) = shapes

    def norm_lrelu(y, s_ref, p, c):
        z = jnp.concatenate([y.astype(jnp.bfloat16),
                             (y * y).astype(jnp.bfloat16)], axis=1)
        st = jnp.dot(s_ref[...], z, preferred_element_type=jnp.float32) / p
        mean = st[:, :c]
        var = st[:, c:] - mean * mean
        scale = jax.lax.rsqrt(var + _EPS)
        y = y.reshape(bsz, p, c)
        yn = (y - mean[:, None, :]) * scale[:, None, :]
        return jnp.where(yn > 0, yn, _SLOPE * yn)

    def store_padded(buf, y, oh, ow, c):
        hp, wp = buf.shape[1], buf.shape[2]
        buf[:, :_PAD] = jnp.zeros_like(buf[:, :_PAD])
        buf[:, _PAD + oh:] = jnp.zeros_like(buf[:, _PAD + oh:])
        buf[:, _PAD:_PAD + oh, :_PAD] = (
            jnp.zeros_like(buf[:, _PAD:_PAD + oh, :_PAD]))
        buf[:, _PAD:_PAD + oh, _PAD + ow:] = (
            jnp.zeros_like(buf[:, _PAD:_PAD + oh, _PAD + ow:]))
        buf[:, _PAD:_PAD + oh, _PAD:_PAD + ow] = (
            y.reshape(bsz, oh, ow, c).astype(buf.dtype))

    def conv_taps(buf, w_ref, cin, stride, oh, ow):
        p = oh * ow
        if stride == 2:
            hph = buf.shape[1] // 2
            wph = buf.shape[2] // 2
            phases = [[buf[:, pl.ds(a, hph, stride=2),
                          pl.ds(b, wph, stride=2), :]
                       for b in range(2)] for a in range(2)]
        y = None
        for i in range(_K):
            for j in range(_K):
                t = i * _K + j
                if stride == 2:
                    xs = phases[i % 2][j % 2][:, i // 2:i // 2 + oh,
                                              j // 2:j // 2 + ow, :]
                else:
                    xs = buf[:, i:i + oh, j:j + ow, :]
                xt = xs.reshape(bsz * p, cin).astype(jnp.bfloat16)
                d = jnp.dot(xt, w_ref[t * cin:(t + 1) * cin, :],
                            preferred_element_type=jnp.float32)
                y = d if y is None else y + d
        return y

    p1 = oh1 * ow1
    y = jnp.dot(cols_ref[...].reshape(bsz * p1, _K * _K * 3), w1_ref[...],
                preferred_element_type=jnp.float32)
    y = norm_lrelu(y, s1_ref, p1, 64)
    store_padded(act1, y, oh1, ow1, 64)

    y = conv_taps(act1, w2_ref, 64, 2, oh2, ow2)
    y = norm_lrelu(y, s2_ref, oh2 * ow2, 128)
    store_padded(act2, y, oh2, ow2, 128)

    y = conv_taps(act2, w3_ref, 128, 2, oh3, ow3)
    y = norm_lrelu(y, s3_ref, oh3 * ow3, 256)
    store_padded(act3, y, oh3, ow3, 256)

    y = conv_taps(act3, w4_ref, 256, 1, oh4, ow4)
    y = norm_lrelu(y, s4_ref, oh4 * ow4, 512)
    store_padded(act4, y, oh4, ow4, 512)

    hp4, wp4 = act4.shape[1], act4.shape[2]
    g = jnp.dot(act4[...].reshape(bsz * hp4 * wp4, 512), w5t_ref[...],
                preferred_element_type=jnp.float32)
    g4 = g.reshape(bsz, hp4, wp4, _K * _K)
    lane = jax.lax.broadcasted_iota(jnp.int32, (1, 1, 1, _K * _K), 3)
    acc = jnp.zeros((bsz, oh5, ow5, _K * _K), jnp.float32)
    for i in range(_K):
        for j in range(_K):
            t = i * _K + j
            acc = acc + jnp.where(lane == t,
                                  g4[:, i:i + oh5, j:j + ow5, :], 0.0)
    y5 = jnp.sum(acc, axis=-1) + b5_ref[0, 0]
    o_ref[...] = jax.nn.sigmoid(y5)


def kernel(x, w1, b1, w2, b2, w3, b3, w4, b4, w5, b5):
    n, cin, h, w = x.shape
    assert cin == 3
    shapes = _spatial(h, w)
    (oh1, ow1), (oh2, ow2), (oh3, ow3), (oh4, ow4), (oh5, ow5) = shapes
    p1 = oh1 * ow1

    xt = jnp.transpose(x, (0, 2, 3, 1))
    xp = jnp.pad(xt, ((0, 0), (_PAD, _PAD), (_PAD, _PAD), (0, 0)))
    taps = []
    for i in range(_K):
        for j in range(_K):
            taps.append(xp[:, i:i + 2 * oh1 - 1:2, j:j + 2 * ow1 - 1:2, :])
    cols = jnp.concatenate(taps, axis=-1).reshape(n, p1, _K * _K * 3)
    cols = cols.astype(jnp.bfloat16)

    w1m = w1.reshape(_K * _K * 3, 64).astype(jnp.bfloat16)
    w2m = w2.reshape(_K * _K * 64, 128).astype(jnp.bfloat16)
    w3m = w3.reshape(_K * _K * 128, 256).astype(jnp.bfloat16)
    w4m = w4.reshape(_K * _K * 256, 512).astype(jnp.bfloat16)
    w5t = jnp.transpose(w5.reshape(_K * _K, 512), (1, 0)).astype(jnp.bfloat16)
    b5s = b5.reshape(1, 1).astype(jnp.float32)

    bsz = max(d for d in (8, 4, 2, 1) if n % d == 0)
    grid = (n // bsz,)

    def sel(p):
        r = jnp.arange(bsz, dtype=jnp.int32)[:, None]
        q = jnp.arange(bsz * p, dtype=jnp.int32)[None, :] // p
        return (r == q).astype(jnp.bfloat16)

    sels = [sel(oh * ow) for (oh, ow) in shapes[:4]]

    scratch_shapes = [
        pltpu.VMEM((bsz, _padded_dim(oh1, True), _padded_dim(ow1, True), 64),
                   jnp.float32),
        pltpu.VMEM((bsz, _padded_dim(oh2, True), _padded_dim(ow2, True), 128),
                   jnp.float32),
        pltpu.VMEM((bsz, _padded_dim(oh3, False), _padded_dim(ow3, False),
                    256), jnp.bfloat16),
        pltpu.VMEM((bsz, _padded_dim(oh4, False), _padded_dim(ow4, False),
                    512), jnp.bfloat16),
    ]

    body = functools.partial(_disc_kernel, bsz=bsz, shapes=tuple(shapes))

    in_specs = (
        [pl.BlockSpec((bsz, p1, _K * _K * 3), lambda i: (i, 0, 0)),
         pl.BlockSpec(w1m.shape, lambda i: (0, 0)),
         pl.BlockSpec(w2m.shape, lambda i: (0, 0)),
         pl.BlockSpec(w3m.shape, lambda i: (0, 0)),
         pl.BlockSpec(w4m.shape, lambda i: (0, 0)),
         pl.BlockSpec(w5t.shape, lambda i: (0, 0)),
         pl.BlockSpec(b5s.shape, lambda i: (0, 0))]
        + [pl.BlockSpec(s.shape, lambda i: (0, 0)) for s in sels])

    cins = [3, 64, 128, 256, 512]
    couts = [64, 128, 256, 512, 16]
    flops = sum(2 * n * oh * ow * (_K * _K * ci) * co
                for (oh, ow), ci, co in zip(shapes, cins, couts))
    transcendentals = n * (64 + 128 + 256 + 512) + n * oh5 * ow5
    bytes_accessed = (cols.size * 2 + w1m.size * 2 + w2m.size * 2
                      + w3m.size * 2 + w4m.size * 2 + w5t.size * 2
                      + n * oh5 * ow5 * 4)

    out = pl.pallas_call(
        body,
        out_shape=jax.ShapeDtypeStruct((n, oh5, ow5), jnp.float32),
        grid=grid,
        in_specs=in_specs,
        out_specs=pl.BlockSpec((bsz, oh5, ow5), lambda i: (i, 0, 0)),
        scratch_shapes=scratch_shapes,
        compiler_params=pltpu.CompilerParams(
            dimension_semantics=("parallel",),
            vmem_limit_bytes=56 * 1024 * 1024,
        ),
        cost_estimate=pl.CostEstimate(
            flops=flops, transcendentals=transcendentals,
            bytes_accessed=bytes_accessed),
    )(cols, w1m, w2m, w3m, w4m, w5t, b5s, *sels)

    return out[:, None, :, :]


# speedup vs baseline: 1.0348x; 1.0344x over previous
import functools

import jax
import jax.numpy as jnp
from jax.experimental import pallas as pl
from jax.experimental.pallas import tpu as pltpu

_K = 4
_PAD = 2
_EPS = 1e-5
_SLOPE = 0.2

_CFGS = [
    (3, 64, 2),
    (64, 128, 2),
    (128, 256, 2),
    (256, 512, 1),
    (512, 1, 1),
]


def _spatial(h, w):
    shapes = []
    ih, iw = h, w
    for (_ci, _co, stride) in _CFGS:
        oh = (ih + 2 * _PAD - _K) // stride + 1
        ow = (iw + 2 * _PAD - _K) // stride + 1
        shapes.append((oh, ow))
        ih, iw = oh, ow
    return shapes


def _padded_dim(o, even):
    d = o + 2 * _PAD
    return d + (d % 2) if even else d


def _disc_kernel(cols_ref, w1_ref, w2_ref, w3_ref, w4_ref, w5t_ref, b5_ref,
                 s1_ref, s2_ref, s3_ref, s4_ref,
                 o_ref,
                 act1, act2, act3, act4,
                 *, bsz, shapes):
    (oh1, ow1), (oh2, ow2), (oh3, ow3), (oh4, ow4), (oh5, ow5) = shapes

    def norm_lrelu(y, s_ref, p, c):
        z = jnp.concatenate([y.astype(jnp.bfloat16),
                             (y * y).astype(jnp.bfloat16)], axis=1)
        st = jnp.dot(s_ref[...], z, preferred_element_type=jnp.float32) / p
        mean = st[:, :c]
        var = st[:, c:] - mean * mean
        scale = jax.lax.rsqrt(var + _EPS)
        y = y.reshape(bsz, p, c)
        yn = (y - mean[:, None, :]) * scale[:, None, :]
        return jnp.where(yn > 0, yn, _SLOPE * yn)

    def store_padded(buf, y, oh, ow, c):
        hp, wp = buf.shape[1], buf.shape[2]
        buf[:, :_PAD] = jnp.zeros_like(buf[:, :_PAD])
        buf[:, _PAD + oh:] = jnp.zeros_like(buf[:, _PAD + oh:])
        buf[:, _PAD:_PAD + oh, :_PAD] = (
            jnp.zeros_like(buf[:, _PAD:_PAD + oh, :_PAD]))
        buf[:, _PAD:_PAD + oh, _PAD + ow:] = (
            jnp.zeros_like(buf[:, _PAD:_PAD + oh, _PAD + ow:]))
        buf[:, _PAD:_PAD + oh, _PAD:_PAD + ow] = (
            y.reshape(bsz, oh, ow, c).astype(buf.dtype))

    def conv_taps(buf, w_ref, cin, stride, oh, ow):
        p = oh * ow
        if stride == 2:
            hph = buf.shape[1] // 2
            wph = buf.shape[2] // 2
            phases = [[buf[:, pl.ds(a, hph, stride=2),
                          pl.ds(b, wph, stride=2), :]
                       for b in range(2)] for a in range(2)]
        y = None
        for i in range(_K):
            for j in range(_K):
                t = i * _K + j
                if stride == 2:
                    xs = phases[i % 2][j % 2][:, i // 2:i // 2 + oh,
                                              j // 2:j // 2 + ow, :]
                else:
                    xs = buf[:, i:i + oh, j:j + ow, :]
                xt = xs.reshape(bsz * p, cin).astype(jnp.bfloat16)
                d = jnp.dot(xt, w_ref[t * cin:(t + 1) * cin, :],
                            preferred_element_type=jnp.float32)
                y = d if y is None else y + d
        return y

    p1 = oh1 * ow1
    y = jnp.dot(cols_ref[...].reshape(bsz * p1, _K * _K * 3), w1_ref[...],
                preferred_element_type=jnp.float32)
    y = norm_lrelu(y, s1_ref, p1, 64)
    store_padded(act1, y, oh1, ow1, 64)

    y = conv_taps(act1, w2_ref, 64, 2, oh2, ow2)
    y = norm_lrelu(y, s2_ref, oh2 * ow2, 128)
    store_padded(act2, y, oh2, ow2, 128)

    y = conv_taps(act2, w3_ref, 128, 2, oh3, ow3)
    y = norm_lrelu(y, s3_ref, oh3 * ow3, 256)
    store_padded(act3, y, oh3, ow3, 256)

    y = conv_taps(act3, w4_ref, 256, 1, oh4, ow4)
    y = norm_lrelu(y, s4_ref, oh4 * ow4, 512)
    store_padded(act4, y, oh4, ow4, 512)

    hp4, wp4 = act4.shape[1], act4.shape[2]
    g = jnp.dot(act4[...].reshape(bsz * hp4 * wp4, 512), w5t_ref[...],
                preferred_element_type=jnp.float32)
    g4 = g.reshape(bsz, hp4, wp4, _K * _K)
    lane = jax.lax.broadcasted_iota(jnp.int32, (1, 1, 1, _K * _K), 3)
    acc = jnp.zeros((bsz, oh5, ow5, _K * _K), jnp.float32)
    for i in range(_K):
        for j in range(_K):
            t = i * _K + j
            acc = acc + jnp.where(lane == t,
                                  g4[:, i:i + oh5, j:j + ow5, :], 0.0)
    y5 = jnp.sum(acc, axis=-1) + b5_ref[0, 0]
    o_ref[...] = jax.nn.sigmoid(y5)


def kernel(x, w1, b1, w2, b2, w3, b3, w4, b4, w5, b5):
    n, cin, h, w = x.shape
    assert cin == 3
    shapes = _spatial(h, w)
    (oh1, ow1), (oh2, ow2), (oh3, ow3), (oh4, ow4), (oh5, ow5) = shapes
    p1 = oh1 * ow1

    xt = jnp.transpose(x, (0, 2, 3, 1))
    xp = jnp.pad(xt, ((0, 0), (_PAD, _PAD), (_PAD, _PAD), (0, 0)))
    taps = []
    for i in range(_K):
        for j in range(_K):
            taps.append(xp[:, i:i + 2 * oh1 - 1:2, j:j + 2 * ow1 - 1:2, :])
    cols = jnp.concatenate(taps, axis=-1).reshape(n, p1, _K * _K * 3)
    cols = cols.astype(jnp.bfloat16)

    w1m = w1.reshape(_K * _K * 3, 64).astype(jnp.bfloat16)
    w2m = w2.reshape(_K * _K * 64, 128).astype(jnp.bfloat16)
    w3m = w3.reshape(_K * _K * 128, 256).astype(jnp.bfloat16)
    w4m = w4.reshape(_K * _K * 256, 512).astype(jnp.bfloat16)
    w5t = jnp.transpose(w5.reshape(_K * _K, 512), (1, 0)).astype(jnp.bfloat16)
    b5s = b5.reshape(1, 1).astype(jnp.float32)

    bsz = max(d for d in (16, 8, 4, 2, 1) if n % d == 0)
    grid = (n // bsz,)

    def sel(p):
        r = jnp.arange(bsz, dtype=jnp.int32)[:, None]
        q = jnp.arange(bsz * p, dtype=jnp.int32)[None, :] // p
        return (r == q).astype(jnp.bfloat16)

    sels = [sel(oh * ow) for (oh, ow) in shapes[:4]]

    scratch_shapes = [
        pltpu.VMEM((bsz, _padded_dim(oh1, True), _padded_dim(ow1, True), 64),
                   jnp.float32),
        pltpu.VMEM((bsz, _padded_dim(oh2, True), _padded_dim(ow2, True), 128),
                   jnp.float32),
        pltpu.VMEM((bsz, _padded_dim(oh3, False), _padded_dim(ow3, False),
                    256), jnp.bfloat16),
        pltpu.VMEM((bsz, _padded_dim(oh4, False), _padded_dim(ow4, False),
                    512), jnp.bfloat16),
    ]

    body = functools.partial(_disc_kernel, bsz=bsz, shapes=tuple(shapes))

    in_specs = (
        [pl.BlockSpec((bsz, p1, _K * _K * 3), lambda i: (i, 0, 0)),
         pl.BlockSpec(w1m.shape, lambda i: (0, 0)),
         pl.BlockSpec(w2m.shape, lambda i: (0, 0)),
         pl.BlockSpec(w3m.shape, lambda i: (0, 0)),
         pl.BlockSpec(w4m.shape, lambda i: (0, 0)),
         pl.BlockSpec(w5t.shape, lambda i: (0, 0)),
         pl.BlockSpec(b5s.shape, lambda i: (0, 0))]
        + [pl.BlockSpec(s.shape, lambda i: (0, 0)) for s in sels])

    cins = [3, 64, 128, 256, 512]
    couts = [64, 128, 256, 512, 16]
    flops = sum(2 * n * oh * ow * (_K * _K * ci) * co
                for (oh, ow), ci, co in zip(shapes, cins, couts))
    transcendentals = n * (64 + 128 + 256 + 512) + n * oh5 * ow5
    bytes_accessed = (cols.size * 2 + w1m.size * 2 + w2m.size * 2
                      + w3m.size * 2 + w4m.size * 2 + w5t.size * 2
                      + n * oh5 * ow5 * 4)

    out = pl.pallas_call(
        body,
        out_shape=jax.ShapeDtypeStruct((n, oh5, ow5), jnp.float32),
        grid=grid,
        in_specs=in_specs,
        out_specs=pl.BlockSpec((bsz, oh5, ow5), lambda i: (i, 0, 0)),
        scratch_shapes=scratch_shapes,
        compiler_params=pltpu.CompilerParams(
            dimension_semantics=("parallel",),
            vmem_limit_bytes=56 * 1024 * 1024,
        ),
        cost_estimate=pl.CostEstimate(
            flops=flops, transcendentals=transcendentals,
            bytes_accessed=bytes_accessed),
    )(cols, w1m, w2m, w3m, w4m, w5t, b5s, *sels)

    return out[:, None, :, :]
